# Initial kernel scaffold; baseline (speedup 1.0000x reference)
import jax
import jax.numpy as jnp
from jax.experimental import pallas as pl


def kernel(h, probs, log_sigma_l):
    raise NotImplementedError("write your pallas kernel here")


# speedup vs baseline: 1.5013x; 1.5013x over previous
import functools

import jax
import jax.numpy as jnp
from jax.experimental import pallas as pl
from jax.experimental.pallas import tpu as pltpu

_B, _N, _D, _K = 1, 8192, 64, 1024
_NB = 1024
_W = 2 * _D + 8


def _fused_kernel(sig_ref, probs_ref, haug_ref, ex_ref,
                  protos_ref, dist_ref, lam_ref, acc_ref):
    i = pl.program_id(0)
    nsteps = pl.num_programs(0)

    @pl.when(i == 0)
    def _init():
        acc_ref[...] = jnp.zeros_like(acc_ref)

    pb = probs_ref[...].astype(jnp.bfloat16)
    hb = haug_ref[...].astype(jnp.bfloat16)
    acc_ref[...] += jax.lax.dot_general(
        pb, hb, dimension_numbers=(((0,), (0,)), ((), ())),
        preferred_element_type=jnp.float32)

    @pl.when(i == nsteps - 1)
    def _epilogue():
        acc = acc_ref[...]
        psum = acc[:, 2 * _D:2 * _D + 1]
        denom = jnp.where(psum == 0.0, 1.0, psum)
        protos = acc[:, :2 * _D] / denom
        pr = protos[:, :_D]
        pi = protos[:, _D:]
        protos_ref[0] = pr
        protos_ref[1] = pi
        mr = jnp.mean(pr, axis=0, keepdims=True)
        mi = jnp.mean(pi, axis=0, keepdims=True)
        rho = jnp.mean((pr - mr) ** 2 + (pi - mi) ** 2)
        sigma = jnp.exp(sig_ref[0])
        lam = jnp.abs(-2.0 * sigma * jnp.log(0.01)
                      + sigma * jnp.log(1.0 + rho / sigma))
        lam_ref[0] = lam
        ex = ex_ref[0:1, :2 * _D]
        dist_ref[...] = jnp.sum((protos - ex) ** 2, axis=1, keepdims=True)


@jax.jit
def kernel(h, probs, log_sigma_l):
    n, k, d = _N, _K, _D
    h2 = h[0].reshape(n, 2 * d)
    haug = jnp.concatenate(
        [h2, jnp.ones((n, 8), dtype=h2.dtype)], axis=1)
    probs2 = probs[0]
    ex = haug[0:1]

    grid = (n // _NB,)
    protos2, dist_col, lam = pl.pallas_call(
        _fused_kernel,
        grid=grid,
        in_specs=[
            pl.BlockSpec(memory_space=pltpu.SMEM),
            pl.BlockSpec((_NB, k), lambda i: (i, 0)),
            pl.BlockSpec((_NB, _W), lambda i: (i, 0)),
            pl.BlockSpec((1, _W), lambda i: (0, 0)),
        ],
        out_specs=[
            pl.BlockSpec((2, k, d), lambda i: (0, 0, 0)),
            pl.BlockSpec((k, 1), lambda i: (0, 0)),
            pl.BlockSpec(memory_space=pltpu.SMEM),
        ],
        out_shape=[
            jax.ShapeDtypeStruct((2, k, d), jnp.float32),
            jax.ShapeDtypeStruct((k, 1), jnp.float32),
            jax.ShapeDtypeStruct((1,), jnp.float32),
        ],
        scratch_shapes=[pltpu.VMEM((k, _W), jnp.float32)],
        compiler_params=pltpu.CompilerParams(
            dimension_semantics=("arbitrary",)),
    )(log_sigma_l, probs2, haug, ex)

    protos = protos2[None]
    dist = dist_col.reshape(1, k)
    lamda = lam.reshape(())
    return (protos, dist, lamda)


# speedup vs baseline: 1.6857x; 1.1228x over previous
import jax
import jax.numpy as jnp
from jax.experimental import pallas as pl
from jax.experimental.pallas import tpu as pltpu

_B, _N, _D, _K = 1, 8192, 64, 1024
_NB = 1024
_W = 2 * _D + 8


def _fused_kernel(sig_ref, probs_ref, haug_ref, ext_ref,
                  protos_ref, dist_ref, lam_ref, acc_ref):
    i = pl.program_id(0)
    nsteps = pl.num_programs(0)

    @pl.when(i == 0)
    def _init():
        acc_ref[...] = jnp.zeros_like(acc_ref)

    pb = probs_ref[...].astype(jnp.bfloat16)
    hb = haug_ref[...].astype(jnp.bfloat16)
    acc_ref[...] += jax.lax.dot_general(
        hb, pb, dimension_numbers=(((0,), (0,)), ((), ())),
        preferred_element_type=jnp.float32)

    @pl.when(i == nsteps - 1)
    def _epilogue():
        acc = acc_ref[...]
        psum = acc[2 * _D:2 * _D + 1, :]
        denom = jnp.where(psum == 0.0, 1.0, psum)
        protos_t = acc[:2 * _D, :] / denom
        pr_t = protos_t[:_D, :]
        pi_t = protos_t[_D:, :]
        protos_ref[0] = pr_t.T
        protos_ref[1] = pi_t.T
        mr = jnp.mean(pr_t, axis=1, keepdims=True)
        mi = jnp.mean(pi_t, axis=1, keepdims=True)
        rho = jnp.mean((pr_t - mr) ** 2 + (pi_t - mi) ** 2)
        sigma = jnp.exp(sig_ref[0])
        lam = jnp.abs(-2.0 * sigma * jnp.log(0.01)
                      + sigma * jnp.log(1.0 + rho / sigma))
        lam_ref[0] = lam
        ext = ext_ref[0:2 * _D, :]
        dist_ref[...] = jnp.sum((protos_t - ext) ** 2, axis=0, keepdims=True)


@jax.jit
def kernel(h, probs, log_sigma_l):
    n, k, d = _N, _K, _D
    h2 = h[0].reshape(n, 2 * d)
    haug = jnp.concatenate(
        [h2, jnp.ones((n, 8), dtype=h2.dtype)], axis=1)
    probs2 = probs[0]
    ext = haug[0].reshape(_W, 1)

    grid = (n // _NB,)
    protos2, dist, lam = pl.pallas_call(
        _fused_kernel,
        grid=grid,
        in_specs=[
            pl.BlockSpec(memory_space=pltpu.SMEM),
            pl.BlockSpec((_NB, k), lambda i: (i, 0)),
            pl.BlockSpec((_NB, _W), lambda i: (i, 0)),
            pl.BlockSpec((_W, 1), lambda i: (0, 0)),
        ],
        out_specs=[
            pl.BlockSpec((2, k, d), lambda i: (0, 0, 0)),
            pl.BlockSpec((1, k), lambda i: (0, 0)),
            pl.BlockSpec(memory_space=pltpu.SMEM),
        ],
        out_shape=[
            jax.ShapeDtypeStruct((2, k, d), jnp.float32),
            jax.ShapeDtypeStruct((1, k), jnp.float32),
            jax.ShapeDtypeStruct((1,), jnp.float32),
        ],
        scratch_shapes=[pltpu.VMEM((_W, k), jnp.float32)],
        compiler_params=pltpu.CompilerParams(
            dimension_semantics=("arbitrary",)),
    )(log_sigma_l, probs2, haug, ext)

    protos = protos2[None]
    lamda = lam.reshape(())
    return (protos, dist, lamda)
